# Initial kernel scaffold; baseline (speedup 1.0000x reference)
import jax
import jax.numpy as jnp
from jax.experimental import pallas as pl


def kernel(ob, body_pos, body_depth, joint_posA, joint_posB, joint_depth, lidar_p2, hull):
    raise NotImplementedError("write your pallas kernel here")


# speedup vs baseline: 2.8492x; 2.8492x over previous
import functools

import jax
import jax.numpy as jnp
from jax import lax
from jax.experimental import pallas as pl
from jax.experimental.pallas import tpu as pltpu
from jax.experimental.pallas import tpu_sc as plsc

GRID_EDGE = 1024
NCELL = 2 * GRID_EDGE * GRID_EDGE
OB_LEN = 64
OUT_LEN = OB_LEN + NCELL
NB, NJ, NL = 1_000_000, 500_000, 100_000
NPTS = NB + 2 * NJ + NL
NW = 32
CHUNK = 2048
TPW = ((NPTS + NW - 1) // NW + CHUNK - 1) // CHUNK * CHUNK
NPAD = NW * TPW
MAGIC = 12582912.0

_mesh = plsc.VectorSubcoreMesh(core_axis_name="c", subcore_axis_name="s")


@functools.partial(
    pl.kernel,
    mesh=_mesh,
    scratch_types=[
        pltpu.VMEM((CHUNK,), jnp.float32),
        pltpu.VMEM((CHUNK,), jnp.float32),
        pltpu.VMEM((CHUNK,), jnp.int32),
        pltpu.VMEM((CHUNK,), jnp.int32),
        pltpu.VMEM((CHUNK,), jnp.float32),
        pltpu.VMEM((16,), jnp.float32),
        pltpu.VMEM((16,), jnp.float32),
    ],
)
def _scatter_grid(xs, ys, ds, zx16, zy16, ones_hbm, out_ref,
                  x_v, y_v, d_v, idx_v, ones_v, zx_v, zy_v):
    wid = lax.axis_index("s") * 2 + lax.axis_index("c")
    base = wid * TPW
    pltpu.sync_copy(zx16, zx_v)
    pltpu.sync_copy(zy16, zy_v)
    pltpu.sync_copy(ones_hbm, ones_v)
    zx = zx_v[...]
    zy = zy_v[...]

    def _to_grid(coord, zero):
        t = (coord - zero) * float(GRID_EDGE)
        t = jnp.minimum(jnp.maximum(t, -1.0), float(GRID_EDGE + 1))
        t = (t + MAGIC) - MAGIC
        t = jnp.minimum(jnp.maximum(t, 0.0), float(GRID_EDGE - 1))
        return t.astype(jnp.int32)

    def chunk_body(ci, carry):
        off = base + ci * CHUNK
        pltpu.sync_copy(xs.at[pl.ds(off, CHUNK)], x_v)
        pltpu.sync_copy(ys.at[pl.ds(off, CHUNK)], y_v)
        pltpu.sync_copy(ds.at[pl.ds(off, CHUNK)], d_v)

        def vec_body(i, c):
            s = i * 16
            ix = _to_grid(x_v[pl.ds(s, 16)], zx)
            iy = _to_grid(y_v[pl.ds(s, 16)], zy)
            dv = d_v[pl.ds(s, 16)]
            flat = ((dv << 20) | (ix << 10) | iy) + OB_LEN
            idx_v[pl.ds(s, 16)] = flat
            return c

        lax.fori_loop(0, CHUNK // 16, vec_body, 0)
        pltpu.sync_copy(ones_v, out_ref.at[idx_v])
        return carry

    lax.fori_loop(0, TPW // CHUNK, chunk_body, 0)


def kernel(ob, body_pos, body_depth, joint_posA, joint_posB, joint_depth,
           lidar_p2, hull):
    zero_x = hull[0] - 0.5
    zero_y = hull[1] - 0.5
    pad = NPAD - NPTS
    pos = jnp.concatenate(
        [body_pos, joint_posA, joint_posB, lidar_p2, body_pos[:pad]], axis=0)
    xs = pos[:, 0]
    ys = pos[:, 1]
    ds = jnp.concatenate(
        [body_depth, joint_depth, joint_depth,
         jnp.zeros((NL,), jnp.int32), body_depth[:pad]])
    zx16 = jnp.full((16,), zero_x, jnp.float32)
    zy16 = jnp.full((16,), zero_y, jnp.float32)
    ones = jnp.ones((CHUNK,), jnp.float32)
    init = jnp.concatenate(
        [ob.reshape(OB_LEN), jnp.zeros((NCELL,), jnp.float32)])
    buf = jax.new_ref(init)
    _scatter_grid(xs, ys, ds, zx16, zy16, ones, buf)
    return buf[...].reshape(1, OUT_LEN)


# speedup vs baseline: 2.8494x; 1.0000x over previous
import functools

import jax
import jax.numpy as jnp
from jax import lax
from jax.experimental import pallas as pl
from jax.experimental.pallas import tpu as pltpu
from jax.experimental.pallas import tpu_sc as plsc

GRID_EDGE = 1024
NCELL = 2 * GRID_EDGE * GRID_EDGE
OB_LEN = 64
OUT_LEN = OB_LEN + NCELL
NB, NJ, NL = 1_000_000, 500_000, 100_000
NPTS = NB + 2 * NJ + NL
NW = 32
CHUNK = 2048
TPW = ((NPTS + NW - 1) // NW + CHUNK - 1) // CHUNK * CHUNK
NPAD = NW * TPW
MAGIC = 12582912.0

_mesh = plsc.VectorSubcoreMesh(core_axis_name="c", subcore_axis_name="s")


@functools.partial(
    pl.kernel,
    mesh=_mesh,
    scratch_types=[
        pltpu.VMEM((CHUNK,), jnp.float32),
        pltpu.VMEM((CHUNK,), jnp.float32),
        pltpu.VMEM((CHUNK,), jnp.int32),
        pltpu.VMEM((CHUNK,), jnp.int32),
        pltpu.VMEM((128,), jnp.float32),
        pltpu.VMEM((16,), jnp.float32),
        pltpu.VMEM((16,), jnp.float32),
        pltpu.SemaphoreType.DMA,
    ],
)
def _scatter_grid(xs, ys, ds, zx16, zy16, ones_hbm, out_ref,
                  x_v, y_v, d_v, idx_v, ones_v, zx_v, zy_v, sem):
    wid = lax.axis_index("s") * 2 + lax.axis_index("c")
    base = wid * TPW
    pltpu.sync_copy(zx16, zx_v)
    pltpu.sync_copy(zy16, zy_v)
    pltpu.sync_copy(ones_hbm, ones_v)
    zx = zx_v[...]
    zy = zy_v[...]

    def _to_grid(coord, zero):
        t = (coord - zero) * float(GRID_EDGE)
        t = jnp.minimum(jnp.maximum(t, -1.0), float(GRID_EDGE + 1))
        t = (t + MAGIC) - MAGIC
        t = jnp.minimum(jnp.maximum(t, 0.0), float(GRID_EDGE - 1))
        return t.astype(jnp.int32)

    def chunk_body(ci, carry):
        off = base + ci * CHUNK
        pltpu.sync_copy(xs.at[pl.ds(off, CHUNK)], x_v)
        pltpu.sync_copy(ys.at[pl.ds(off, CHUNK)], y_v)
        pltpu.sync_copy(ds.at[pl.ds(off, CHUNK)], d_v)

        def vec_body(i, c):
            s = i * 16
            ix = _to_grid(x_v[pl.ds(s, 16)], zx)
            iy = _to_grid(y_v[pl.ds(s, 16)], zy)
            dv = d_v[pl.ds(s, 16)]
            flat = ((dv << 20) | (ix << 10) | iy) + OB_LEN
            idx_v[pl.ds(s, 16)] = flat
            return c

        lax.fori_loop(0, CHUNK // 16, vec_body, 0)
        for j in range(CHUNK // 128):
            pltpu.sync_copy(ones_v, out_ref.at[idx_v.at[pl.ds(j * 128, 128)]])
        return carry

    lax.fori_loop(0, TPW // CHUNK, chunk_body, 0)


def kernel(ob, body_pos, body_depth, joint_posA, joint_posB, joint_depth,
           lidar_p2, hull):
    zero_x = hull[0] - 0.5
    zero_y = hull[1] - 0.5
    pad = NPAD - NPTS
    pos = jnp.concatenate(
        [body_pos, joint_posA, joint_posB, lidar_p2, body_pos[:pad]], axis=0)
    xs = pos[:, 0]
    ys = pos[:, 1]
    ds = jnp.concatenate(
        [body_depth, joint_depth, joint_depth,
         jnp.zeros((NL,), jnp.int32), body_depth[:pad]])
    zx16 = jnp.full((16,), zero_x, jnp.float32)
    zy16 = jnp.full((16,), zero_y, jnp.float32)
    ones = jnp.ones((128,), jnp.float32)
    init = jnp.concatenate(
        [ob.reshape(OB_LEN), jnp.zeros((NCELL,), jnp.float32)])
    buf = jax.new_ref(init)
    _scatter_grid(xs, ys, ds, zx16, zy16, ones, buf)
    return buf[...].reshape(1, OUT_LEN)
